# Initial kernel scaffold; baseline (speedup 1.0000x reference)
import jax
import jax.numpy as jnp
from jax import lax
from jax.experimental import pallas as pl
from jax.experimental.pallas import tpu as pltpu
from jax.experimental.pallas import tpu_sc as plsc

_VOCAB = 100000
_TAB = _VOCAB + 1
_B, _L = 4096, 200
_MAX_OOV = 51
_NC, _NS, _LANES = 2, 16, 16
_NW = _NC * _NS
_ROWS_W = _B // _NW
_CHUNK = 32
_NCHUNK = _ROWS_W // _CHUNK
_RG = _CHUNK // _LANES


def _body(in_hbm, oovs_hbm, tab_hbm, words_hbm, mask_hbm,
          tab_v, in_v, oov_v, w_v, m_v):
    wid = lax.axis_index("s") * _NC + lax.axis_index("c")
    pltpu.sync_copy(tab_hbm, tab_v)
    lane = lax.iota(jnp.int32, _LANES)
    for c in range(_NCHUNK):
        row0 = wid * _ROWS_W + c * _CHUNK
        pltpu.sync_copy(in_hbm.at[pl.ds(row0, _CHUNK), :], in_v)
        pltpu.sync_copy(oovs_hbm.at[pl.ds(row0, _CHUNK), :], oov_v)
        for rg in range(_RG):
            row_idx = lane + (rg * _LANES)

            def col_body(l, alive, row_idx=row_idx):
                col = jnp.full((_LANES,), l, dtype=jnp.int32)
                ids = plsc.load_gather(in_v, [row_idx, col])
                alive = jnp.where(ids == 1, 0, alive)
                mask_f = alive.astype(jnp.float32)
                tabw = plsc.load_gather(tab_v, [jnp.minimum(ids, _VOCAB)])
                is_oov = ids > _VOCAB
                oov_col = jnp.where(is_oov, ids - _VOCAB, 0)
                oovw = plsc.load_gather(oov_v, [row_idx, oov_col])
                w = jnp.where(is_oov, oovw, tabw)
                w = jnp.where(alive == 0, 0.0, w)
                plsc.store_scatter(w_v, [row_idx, col], w)
                plsc.store_scatter(m_v, [row_idx, col], mask_f)
                return alive

            lax.fori_loop(0, _L, col_body, jnp.ones((_LANES,), jnp.int32))
        pltpu.sync_copy(w_v, words_hbm.at[pl.ds(row0, _CHUNK), :])
        pltpu.sync_copy(m_v, mask_hbm.at[pl.ds(row0, _CHUNK), :])


def kernel(input_seqs, oovs, table):
    mesh = plsc.VectorSubcoreMesh(core_axis_name="c", subcore_axis_name="s")
    f = pl.kernel(
        _body,
        out_type=(
            jax.ShapeDtypeStruct((_B, _L), jnp.float32),
            jax.ShapeDtypeStruct((_B, _L), jnp.float32),
        ),
        mesh=mesh,
        scratch_types=[
            pltpu.VMEM((_TAB,), jnp.float32),
            pltpu.VMEM((_CHUNK, _L), jnp.int32),
            pltpu.VMEM((_CHUNK, _MAX_OOV), jnp.float32),
            pltpu.VMEM((_CHUNK, _L), jnp.float32),
            pltpu.VMEM((_CHUNK, _L), jnp.float32),
        ],
    )
    words, mask = f(input_seqs, oovs, table)
    return (words, mask)


# speedup vs baseline: 212.9464x; 212.9464x over previous
import jax
import jax.numpy as jnp
from jax import lax
from jax.experimental import pallas as pl
from jax.experimental.pallas import tpu as pltpu
from jax.experimental.pallas import tpu_sc as plsc

_VOCAB = 100000
_TAB = _VOCAB + 1
_B, _L = 4096, 200
_MAX_OOV = 51
_NC, _NS, _LANES = 2, 16, 16
_NW = _NC * _NS
_ROWS_W = _B // _NW
_CHUNK = 32
_NCHUNK = _ROWS_W // _CHUNK
_RG = _CHUNK // _LANES


def _body(in_hbm, oovs_hbm, tab_hbm, words_hbm, mask_hbm,
          tab_v, in_v, oov_v, w_v, m_v):
    wid = lax.axis_index("s") * _NC + lax.axis_index("c")
    pltpu.sync_copy(tab_hbm, tab_v)
    lane = lax.iota(jnp.int32, _LANES)
    for c in range(_NCHUNK):
        row0 = wid * _ROWS_W + c * _CHUNK
        pltpu.sync_copy(in_hbm.at[pl.ds(row0, _CHUNK), :], in_v)
        pltpu.sync_copy(oovs_hbm.at[pl.ds(row0, _CHUNK), :], oov_v)
        for rg in range(_RG):
            row_idx = lane + (rg * _LANES)

            def col_body(l, alive, row_idx=row_idx):
                col = jnp.full((_LANES,), l, dtype=jnp.int32)
                ids = plsc.load_gather(in_v, [row_idx, col])
                alive = jnp.where(ids == 1, 0, alive)
                mask_f = alive.astype(jnp.float32)
                tabw = plsc.load_gather(tab_v, [jnp.minimum(ids, _VOCAB)])
                is_oov = ids > _VOCAB
                oov_col = jnp.where(is_oov, ids - _VOCAB, 0)
                oovw = plsc.load_gather(oov_v, [row_idx, oov_col])
                w = jnp.where(is_oov, oovw, tabw)
                w = jnp.where(alive == 0, 0.0, w)
                plsc.store_scatter(w_v, [row_idx, col], w)
                plsc.store_scatter(m_v, [row_idx, col], mask_f)
                return alive

            lax.fori_loop(0, _L, col_body, jnp.ones((_LANES,), jnp.int32))
        pltpu.sync_copy(w_v, words_hbm.at[pl.ds(row0, _CHUNK), :])
        pltpu.sync_copy(m_v, mask_hbm.at[pl.ds(row0, _CHUNK), :])


def kernel(input_seqs, oovs, table):
    mesh = plsc.VectorSubcoreMesh(core_axis_name="c", subcore_axis_name="s")
    f = pl.kernel(
        _body,
        out_type=(
            jax.ShapeDtypeStruct((_B, _L), jnp.float32),
            jax.ShapeDtypeStruct((_B, _L), jnp.float32),
        ),
        mesh=mesh,
        compiler_params=pltpu.CompilerParams(
            use_tc_tiling_on_sc=False, needs_layout_passes=False),
        scratch_types=[
            pltpu.VMEM((_TAB,), jnp.float32),
            pltpu.VMEM((_CHUNK, _L), jnp.int32),
            pltpu.VMEM((_CHUNK, _MAX_OOV), jnp.float32),
            pltpu.VMEM((_CHUNK, _L), jnp.float32),
            pltpu.VMEM((_CHUNK, _L), jnp.float32),
        ],
    )
    words, mask = f(input_seqs, oovs, table)
    return (words, mask)


# speedup vs baseline: 418.1508x; 1.9636x over previous
import jax
import jax.numpy as jnp
from jax import lax
from jax.experimental import pallas as pl
from jax.experimental.pallas import tpu as pltpu
from jax.experimental.pallas import tpu_sc as plsc

_VOCAB = 100000
_TAB = _VOCAB + 1
_B, _L = 4096, 200
_MAX_OOV = 51
_NC, _NS, _LANES = 2, 16, 16
_NW = _NC * _NS
_COLS_W = _B // _NW
_NG = _COLS_W // _LANES
_LCHUNK = 40
_NLCHUNK = _L // _LCHUNK


def _body(in_hbm, oovs_hbm, tab_hbm, words_hbm, mask_hbm,
          tab_v, in_v, oov_v, w_v, m_v):
    wid = lax.axis_index("s") * _NC + lax.axis_index("c")
    c0 = wid * _COLS_W
    pltpu.sync_copy(tab_hbm, tab_v)
    pltpu.sync_copy(oovs_hbm.at[:, pl.ds(c0, _COLS_W)], oov_v)
    lane = lax.iota(jnp.int32, _LANES)
    lane_cols = [lane + g * _LANES for g in range(_NG)]
    alives = tuple(jnp.ones((_LANES,), jnp.int32) for _ in range(_NG))
    for k in range(_NLCHUNK):
        l0 = k * _LCHUNK
        pltpu.sync_copy(in_hbm.at[pl.ds(l0, _LCHUNK), pl.ds(c0, _COLS_W)],
                        in_v)

        def lbody(l, alives):
            new = []
            for g in range(_NG):
                ids = in_v[l, pl.ds(g * _LANES, _LANES)]
                alive = jnp.where(ids == 1, 0, alives[g])
                mask_f = alive.astype(jnp.float32)
                tabw = plsc.load_gather(tab_v, [jnp.minimum(ids, _VOCAB)])
                is_oov = ids > _VOCAB
                oov_row = jnp.where(is_oov, ids - _VOCAB, 0)
                oovw = plsc.load_gather(oov_v, [oov_row, lane_cols[g]])
                w = jnp.where(is_oov, oovw, tabw)
                w = jnp.where(alive == 0, 0.0, w)
                w_v[l, pl.ds(g * _LANES, _LANES)] = w
                m_v[l, pl.ds(g * _LANES, _LANES)] = mask_f
                new.append(alive)
            return tuple(new)

        alives = lax.fori_loop(0, _LCHUNK, lbody, alives)
        pltpu.sync_copy(w_v, words_hbm.at[pl.ds(l0, _LCHUNK),
                                          pl.ds(c0, _COLS_W)])
        pltpu.sync_copy(m_v, mask_hbm.at[pl.ds(l0, _LCHUNK),
                                         pl.ds(c0, _COLS_W)])


def kernel(input_seqs, oovs, table):
    mesh = plsc.VectorSubcoreMesh(core_axis_name="c", subcore_axis_name="s")
    f = pl.kernel(
        _body,
        out_type=(
            jax.ShapeDtypeStruct((_L, _B), jnp.float32),
            jax.ShapeDtypeStruct((_L, _B), jnp.float32),
        ),
        mesh=mesh,
        compiler_params=pltpu.CompilerParams(
            use_tc_tiling_on_sc=True, needs_layout_passes=False),
        scratch_types=[
            pltpu.VMEM((_TAB,), jnp.float32),
            pltpu.VMEM((_LCHUNK, _COLS_W), jnp.int32),
            pltpu.VMEM((_MAX_OOV, _COLS_W), jnp.float32),
            pltpu.VMEM((_LCHUNK, _COLS_W), jnp.float32),
            pltpu.VMEM((_LCHUNK, _COLS_W), jnp.float32),
        ],
    )
    words_t, mask_t = f(input_seqs.T, oovs.T, table)
    return (words_t.T, mask_t.T)
